# baseline (speedup 1.0000x reference)
import jax
import jax.numpy as jnp
from jax.experimental import pallas as pl
from jax.experimental.pallas import tpu as pltpu
from jax.experimental.pallas import tpu_sc as plsc

BLK = 64


def _sc_gather(data, idx, subrows):
    n_rows, row_dim = data.shape
    value_dim = row_dim // subrows
    data_v = data.reshape(n_rows * subrows, value_dim)
    n = idx.shape[0]
    idx_v = (
        idx[:, None] * subrows + jnp.arange(subrows, dtype=jnp.int32)[None, :]
    ).reshape(-1)
    out = _sc_gather_raw(data_v, idx_v, 128)
    return out.reshape(n, row_dim)


def _sc_gather_raw(data, idx, window):
    n_out = idx.shape[0]
    value_dim = data.shape[1]
    idx2 = idx.reshape(1, n_out)
    mesh = plsc.VectorSubcoreMesh(core_axis_name="c", subcore_axis_name="s")

    @pl.kernel(
        out_type=jax.ShapeDtypeStruct((n_out, value_dim), data.dtype),
        mesh=mesh,
    )
    def gather_kernel(x_hbm, i_hbm, o_hbm):
        def body(i_vmem, o_vmem):
            pltpu.sync_copy(x_hbm.at[i_vmem.at[0]], o_vmem)

        pltpu.emit_pipeline(
            body,
            grid=(n_out // window,),
            in_specs=[pl.BlockSpec((1, window), index_map=lambda i: (0, i))],
            out_specs=[pl.BlockSpec((window, value_dim), index_map=lambda i: (i, 0))],
            core_axis_name=("c", "s"),
            dimension_semantics=(pltpu.PARALLEL,),
        )(i_hbm, o_hbm)

    return gather_kernel(data, idx2)


def _mlp_body(be_ref, x_ref, w1_ref, b1_ref, w2_ref, b2_ref, o_ref):
    h = jnp.maximum(
        jnp.dot(x_ref[...], w1_ref[0], preferred_element_type=jnp.float32)
        + b1_ref[0],
        0.0,
    )
    o = jnp.dot(h, w2_ref[0], preferred_element_type=jnp.float32) + b2_ref[0]
    norm = jnp.sqrt(jnp.sum(o * o, axis=1, keepdims=True))
    o_ref[...] = o / jnp.maximum(norm, 1e-12)


def kernel(eeg_emb, subject_ids, W1, b1, W2, b2):
    B, eeg_dim = eeg_emb.shape
    S, _, clip_dim = W1.shape
    NB = B // BLK + S + 1
    P = NB * BLK

    sid32 = subject_ids.astype(jnp.int32)
    onehot = jax.nn.one_hot(sid32, S, dtype=jnp.int32)
    counts = jnp.sum(onehot, axis=0)
    csum = jnp.cumsum(onehot, axis=0)
    occ = jnp.take_along_axis(csum, sid32[:, None], axis=1)[:, 0] - 1
    blocks_per = (counts + BLK - 1) // BLK
    cb = jnp.cumsum(blocks_per)
    block_start = cb - blocks_per
    total_blocks = cb[-1]
    dest = jnp.take(block_start, sid32) * BLK + occ
    src_for_slot = jnp.zeros(P, jnp.int32).at[dest].set(
        jnp.arange(B, dtype=jnp.int32)
    )
    karr = jnp.arange(NB, dtype=jnp.int32)
    be_arr = jnp.searchsorted(
        cb, jnp.minimum(karr, total_blocks - 1), side="right"
    ).astype(jnp.int32)

    x_sorted = _sc_gather(eeg_emb, src_for_slot, 2)

    b1r = b1.reshape(S, 1, clip_dim)
    b2r = b2.reshape(S, 1, clip_dim)
    grid_spec = pltpu.PrefetchScalarGridSpec(
        num_scalar_prefetch=1,
        grid=(NB,),
        in_specs=[
            pl.BlockSpec((BLK, eeg_dim), lambda i, be: (i, 0)),
            pl.BlockSpec((1, eeg_dim, clip_dim), lambda i, be: (be[i], 0, 0)),
            pl.BlockSpec((1, 1, clip_dim), lambda i, be: (be[i], 0, 0)),
            pl.BlockSpec((1, clip_dim, clip_dim), lambda i, be: (be[i], 0, 0)),
            pl.BlockSpec((1, 1, clip_dim), lambda i, be: (be[i], 0, 0)),
        ],
        out_specs=pl.BlockSpec((BLK, clip_dim), lambda i, be: (i, 0)),
    )
    o_sorted = pl.pallas_call(
        _mlp_body,
        grid_spec=grid_spec,
        out_shape=jax.ShapeDtypeStruct((P, clip_dim), jnp.float32),
    )(be_arr, x_sorted, W1, b1r, W2, b2r)

    out = _sc_gather(o_sorted, dest, 4)
    return out
